# baseline (speedup 1.0000x reference)
import functools

import jax
import jax.numpy as jnp
from jax.experimental import pallas as pl
from jax.experimental.pallas import tpu as pltpu

D_MODEL = 768
D_BASE = 256
NUM_EXPERTS = 64
D_TASK = 32
RANK = 32
TOP_K = 2

BLK = 2048


def _fused(tid_ref, x_ref, wcat_ref, wtask_ref, rb_ref, temb_ref, basis_ref,
           sstr_ref, sidx_ref, out_ref, stats_ref, cnt_ref, *, spb, n_elems):
    p = pl.program_id(0)
    i = pl.program_id(1)
    b = i // spb

    @pl.when(p == 0)
    def _phase0():
        @pl.when(i == 0)
        def _init():
            stats_ref[...] = jnp.zeros_like(stats_ref)
            d_iota = jax.lax.broadcasted_iota(
                jnp.int32, (NUM_EXPERTS, D_BASE), 1)
            cnt = jnp.zeros((NUM_EXPERTS, D_BASE), dtype=jnp.float32)
            for j in range(RANK):
                cnt += (sidx_ref[:, j:j + 1] == d_iota).astype(jnp.float32)
            cnt_ref[...] = cnt

        xb = x_ref[...]
        rows = jax.lax.broadcasted_iota(jnp.int32, stats_ref.shape, 0)
        cols = jax.lax.broadcasted_iota(jnp.int32, stats_ref.shape, 1)
        s = jnp.sum(xb)
        ss = jnp.sum(xb * xb)
        upd = jnp.where(cols == 0, s, jnp.where(cols == 1, ss, 0.0))
        stats_ref[...] += jnp.where(rows == b, upd, 0.0)

    @pl.when(p == 1)
    def _phase1():
        sum_b = stats_ref[pl.ds(b, 1), 0:1]
        ssq_b = stats_ref[pl.ds(b, 1), 1:2]
        mu = sum_b / n_elems
        var = ssq_b / n_elems - mu * mu

        xb = x_ref[...]
        xn = (xb - mu) / jnp.sqrt(var + 1e-5)

        yb = jnp.dot(xn.astype(jnp.bfloat16), wcat_ref[...],
                     preferred_element_type=jnp.float32)
        t_row = temb_ref[pl.ds(tid_ref[b], 1), :]
        tl = jnp.dot(t_row.astype(jnp.bfloat16), wtask_ref[...],
                     preferred_element_type=jnp.float32)
        logits = yb[:, D_BASE:D_BASE + NUM_EXPERTS] + tl + rb_ref[...]

        e_iota = jax.lax.broadcasted_iota(
            jnp.int32, logits.shape, 1).astype(jnp.float32)
        m1 = jnp.max(logits, axis=1, keepdims=True)
        i1 = jnp.min(jnp.where(logits == m1, e_iota, float(NUM_EXPERTS)),
                     axis=1, keepdims=True)
        masked = jnp.where(e_iota == i1, -jnp.inf, logits)
        m2 = jnp.max(masked, axis=1, keepdims=True)
        i2 = jnp.min(jnp.where(masked == m2, e_iota, float(NUM_EXPERTS)),
                     axis=1, keepdims=True)
        z = jnp.exp(m2 - m1)
        w1 = 1.0 / (1.0 + z)
        w2 = z * w1

        gate = jnp.mean(sstr_ref[...], axis=1)[None, :]
        pmat = w1 * (e_iota == i1) + w2 * (e_iota == i2)
        pg = pmat * gate

        mask = jnp.dot(pg.astype(jnp.bfloat16),
                       cnt_ref[...].astype(jnp.bfloat16),
                       preferred_element_type=jnp.float32)
        csum_b = jnp.sum(wcat_ref[:, 0:D_BASE].astype(jnp.float32), axis=0,
                         keepdims=True)
        sigma = jnp.sqrt(var + 1e-5)
        coeff = yb[:, 0:D_BASE] * sigma + mu * csum_b
        out_ref[...] = jnp.dot((coeff * mask).astype(jnp.bfloat16),
                               basis_ref[...],
                               preferred_element_type=jnp.float32)


def kernel(x, task_id, task_embedding, router_W, router_b, basis,
           subspace_strength, subspace_indices):
    Bs, Ss, D = x.shape
    T = Bs * Ss
    n_elems = float(Ss * D)

    xf = x.reshape(T, D)
    rb2 = router_b.reshape(1, NUM_EXPERTS)
    tid = task_id.astype(jnp.int32)
    sidx = subspace_indices.astype(jnp.int32)
    wcat_bf = jnp.concatenate(
        [basis.T, router_W[:D, :]], axis=1).astype(jnp.bfloat16)
    wtask_bf = router_W[D:, :].astype(jnp.bfloat16)
    basis_bf = basis.astype(jnp.bfloat16)

    grid_spec = pltpu.PrefetchScalarGridSpec(
        num_scalar_prefetch=1,
        grid=(2, T // BLK),
        in_specs=[
            pl.BlockSpec((BLK, D), lambda p, i, tid_ref: (i, 0)),
            pl.BlockSpec((D, D_BASE + NUM_EXPERTS),
                         lambda p, i, tid_ref: (0, 0)),
            pl.BlockSpec((D_TASK, NUM_EXPERTS), lambda p, i, tid_ref: (0, 0)),
            pl.BlockSpec((1, NUM_EXPERTS), lambda p, i, tid_ref: (0, 0)),
            pl.BlockSpec(task_embedding.shape, lambda p, i, tid_ref: (0, 0)),
            pl.BlockSpec((D_BASE, D), lambda p, i, tid_ref: (0, 0)),
            pl.BlockSpec((NUM_EXPERTS, RANK), lambda p, i, tid_ref: (0, 0)),
            pl.BlockSpec((NUM_EXPERTS, RANK), lambda p, i, tid_ref: (0, 0)),
        ],
        out_specs=pl.BlockSpec((BLK, D), lambda p, i, tid_ref: (i * p, 0)),
        scratch_shapes=[
            pltpu.VMEM((8, 128), jnp.float32),
            pltpu.VMEM((NUM_EXPERTS, D_BASE), jnp.float32),
        ],
    )
    out = pl.pallas_call(
        functools.partial(_fused, spb=Ss // BLK, n_elems=n_elems),
        grid_spec=grid_spec,
        out_shape=jax.ShapeDtypeStruct((T, D), jnp.float32),
        compiler_params=pltpu.CompilerParams(
            dimension_semantics=("arbitrary", "arbitrary")),
    )(tid, xf, wcat_bf, wtask_bf, rb2, task_embedding, basis_bf,
      subspace_strength, sidx)

    return out.reshape(Bs, Ss, D)


# speedup vs baseline: 1.0281x; 1.0281x over previous
import functools

import jax
import jax.numpy as jnp
from jax.experimental import pallas as pl
from jax.experimental.pallas import tpu as pltpu

D_MODEL = 768
D_BASE = 256
NUM_EXPERTS = 64
D_TASK = 32
RANK = 32
TOP_K = 2

BLK = 2048


def _fused(tid_ref, x_ref, rw_ref, rb_ref, temb_ref, basis_ref, sstr_ref,
           sidx_ref, out_ref, stats_ref, cnt_ref, *, spb, n_elems):
    p = pl.program_id(0)
    i = pl.program_id(1)
    b = i // spb

    @pl.when(p == 0)
    def _phase0():
        @pl.when(i == 0)
        def _init():
            stats_ref[...] = jnp.zeros_like(stats_ref)
            d_iota = jax.lax.broadcasted_iota(
                jnp.int32, (NUM_EXPERTS, D_BASE), 1)
            cnt = jnp.zeros((NUM_EXPERTS, D_BASE), dtype=jnp.float32)
            for j in range(RANK):
                cnt += (sidx_ref[:, j:j + 1] == d_iota).astype(jnp.float32)
            cnt_ref[...] = cnt

        xb = x_ref[...]
        rows = jax.lax.broadcasted_iota(jnp.int32, stats_ref.shape, 0)
        cols = jax.lax.broadcasted_iota(jnp.int32, stats_ref.shape, 1)
        s = jnp.sum(xb)
        ss = jnp.sum(xb * xb)
        upd = jnp.where(cols == 0, s, jnp.where(cols == 1, ss, 0.0))
        stats_ref[...] += jnp.where(rows == b, upd, 0.0)

    @pl.when(p == 1)
    def _phase1():
        sum_b = stats_ref[pl.ds(b, 1), 0:1]
        ssq_b = stats_ref[pl.ds(b, 1), 1:2]
        mu = sum_b / n_elems
        var = ssq_b / n_elems - mu * mu

        xb = x_ref[...]
        xn = (xb - mu) / jnp.sqrt(var + 1e-5)

        logits = jnp.dot(xn.astype(jnp.bfloat16),
                         rw_ref[0:D_MODEL, :].astype(jnp.bfloat16),
                         preferred_element_type=jnp.float32)
        t_row = temb_ref[pl.ds(tid_ref[b], 1), :]
        tl = jnp.dot(t_row.astype(jnp.bfloat16),
                     rw_ref[D_MODEL:D_MODEL + D_TASK, :].astype(jnp.bfloat16),
                     preferred_element_type=jnp.float32)
        logits = logits + tl + rb_ref[...]

        e_iota = jax.lax.broadcasted_iota(
            jnp.int32, logits.shape, 1).astype(jnp.float32)
        m1 = jnp.max(logits, axis=1, keepdims=True)
        i1 = jnp.min(jnp.where(logits == m1, e_iota, float(NUM_EXPERTS)),
                     axis=1, keepdims=True)
        masked = jnp.where(e_iota == i1, -jnp.inf, logits)
        m2 = jnp.max(masked, axis=1, keepdims=True)
        i2 = jnp.min(jnp.where(masked == m2, e_iota, float(NUM_EXPERTS)),
                     axis=1, keepdims=True)
        z = jnp.exp(m2 - m1)
        w1 = 1.0 / (1.0 + z)
        w2 = z * w1

        gate = jnp.mean(sstr_ref[...], axis=1)[None, :]
        pmat = w1 * (e_iota == i1) + w2 * (e_iota == i2)
        pg = pmat * gate

        mask = jnp.dot(pg.astype(jnp.bfloat16),
                       cnt_ref[...].astype(jnp.bfloat16),
                       preferred_element_type=jnp.float32)
        basis_bf = basis_ref[...].astype(jnp.bfloat16)
        coeff = jax.lax.dot_general(
            xb.astype(jnp.bfloat16), basis_bf, (((1,), (1,)), ((), ())),
            preferred_element_type=jnp.float32)
        out_ref[...] = jnp.dot((coeff * mask).astype(jnp.bfloat16), basis_bf,
                               preferred_element_type=jnp.float32)


def kernel(x, task_id, task_embedding, router_W, router_b, basis,
           subspace_strength, subspace_indices):
    Bs, Ss, D = x.shape
    T = Bs * Ss
    n_elems = float(Ss * D)

    xf = x.reshape(T, D)
    rb2 = router_b.reshape(1, NUM_EXPERTS)
    tid = task_id.astype(jnp.int32)
    sidx = subspace_indices.astype(jnp.int32)

    grid_spec = pltpu.PrefetchScalarGridSpec(
        num_scalar_prefetch=1,
        grid=(2, T // BLK),
        in_specs=[
            pl.BlockSpec((BLK, D), lambda p, i, tid_ref: (i, 0)),
            pl.BlockSpec((D + D_TASK, NUM_EXPERTS),
                         lambda p, i, tid_ref: (0, 0)),
            pl.BlockSpec((1, NUM_EXPERTS), lambda p, i, tid_ref: (0, 0)),
            pl.BlockSpec(task_embedding.shape, lambda p, i, tid_ref: (0, 0)),
            pl.BlockSpec((D_BASE, D), lambda p, i, tid_ref: (0, 0)),
            pl.BlockSpec((NUM_EXPERTS, RANK), lambda p, i, tid_ref: (0, 0)),
            pl.BlockSpec((NUM_EXPERTS, RANK), lambda p, i, tid_ref: (0, 0)),
        ],
        out_specs=pl.BlockSpec((BLK, D), lambda p, i, tid_ref: (i * p, 0)),
        scratch_shapes=[
            pltpu.VMEM((8, 128), jnp.float32),
            pltpu.VMEM((NUM_EXPERTS, D_BASE), jnp.float32),
        ],
    )
    out = pl.pallas_call(
        functools.partial(_fused, spb=Ss // BLK, n_elems=n_elems),
        grid_spec=grid_spec,
        out_shape=jax.ShapeDtypeStruct((T, D), jnp.float32),
        compiler_params=pltpu.CompilerParams(
            dimension_semantics=("arbitrary", "arbitrary")),
    )(tid, xf, router_W, rb2, task_embedding, basis,
      subspace_strength, sidx)

    return out.reshape(Bs, Ss, D)
